# baseline (device time: 139538 ns/iter reference)
import jax
import jax.numpy as jnp
from jax import lax
from jax.experimental import pallas as pl
from jax.experimental.pallas import tpu as pltpu

N_DEV = 32
BLK = 128


def kernel(x, w_mat, scale_x, scale_w):
    m_glob, k_loc = x.shape
    k_glob, n = w_mat.shape

    def body(x_ref, w_ref, sx_ref, sw_ref, out_ref,
             recv_buf, send_sems, recv_sems):
        my = lax.axis_index("i")
        left = lax.rem(my + (N_DEV - 1), N_DEV)

        barrier_sem = pltpu.get_barrier_semaphore()
        pl.semaphore_signal(
            barrier_sem, inc=1,
            device_id=(left,), device_id_type=pl.DeviceIdType.MESH,
        )

        recv_buf[pl.ds(0, 1), :, :] = x_ref[pl.ds(my * BLK, BLK), :][None]
        acc = lax.dot_general(
            recv_buf[0], w_ref[pl.ds(my * BLK, BLK), :],
            (((1,), (0,)), ((), ())),
            preferred_element_type=jnp.int32,
        )

        sends = []
        for off in range(1, N_DEV):
            pl.semaphore_wait(barrier_sem, 1)
            if off < N_DEV - 1:
                pl.semaphore_signal(
                    barrier_sem, inc=1,
                    device_id=(left,), device_id_type=pl.DeviceIdType.MESH,
                )
            dst = lax.rem(my + off, N_DEV)
            rdma = pltpu.make_async_remote_copy(
                src_ref=x_ref.at[pl.ds(dst * BLK, BLK), :],
                dst_ref=recv_buf.at[off],
                send_sem=send_sems.at[off],
                recv_sem=recv_sems.at[off],
                device_id=(dst,),
                device_id_type=pl.DeviceIdType.MESH,
            )
            rdma.start()
            sends.append(rdma)

            recv = pltpu.make_async_remote_copy(
                src_ref=recv_buf.at[off],
                dst_ref=recv_buf.at[off],
                send_sem=send_sems.at[0],
                recv_sem=recv_sems.at[off],
                device_id=(my,),
                device_id_type=pl.DeviceIdType.MESH,
            )
            recv.wait_recv()
            src_blk = lax.rem(my + (N_DEV - off), N_DEV)
            acc = acc + lax.dot_general(
                recv_buf[off], w_ref[pl.ds(src_blk * BLK, BLK), :],
                (((1,), (0,)), ((), ())),
                preferred_element_type=jnp.int32,
            )

        scale = sx_ref[0, 0] * sw_ref[0, 0]
        out_ref[...] = acc.astype(jnp.float32) * scale

        for r in sends:
            r.wait_send()

    return pl.pallas_call(
        body,
        out_shape=jax.ShapeDtypeStruct((BLK, n), jnp.float32),
        in_specs=[
            pl.BlockSpec(memory_space=pltpu.VMEM),
            pl.BlockSpec(memory_space=pltpu.VMEM),
            pl.BlockSpec(memory_space=pltpu.SMEM),
            pl.BlockSpec(memory_space=pltpu.SMEM),
        ],
        out_specs=pl.BlockSpec(memory_space=pltpu.VMEM),
        scratch_shapes=[
            pltpu.VMEM((N_DEV, BLK, BLK), jnp.int8),
            pltpu.SemaphoreType.DMA((N_DEV,)),
            pltpu.SemaphoreType.DMA((N_DEV,)),
        ],
        compiler_params=pltpu.CompilerParams(
            vmem_limit_bytes=100 * 1024 * 1024,
            collective_id=0,
        ),
    )(x, w_mat, scale_x.reshape(1, 1), scale_w.reshape(1, 1))
